# baseline (speedup 1.0000x reference)
import functools

import jax
import jax.numpy as jnp
from jax import lax
from jax.experimental import pallas as pl
from jax.experimental.pallas import tpu as pltpu
from jax.experimental.pallas import tpu_sc as plsc

_B = 16384
_D = 128
_NC, _NS, _L = 2, 16, 16
_NW = _NC * _NS
_RPW = _B // _NW
_G = _RPW // _L


def _sc_body(z_hbm, y0_hbm, g_hbm, out_hbm, z_v, g_v, idx_v, out_v):
    wid = lax.axis_index("s") * _NC + lax.axis_index("c")
    base = wid * _RPW
    pltpu.sync_copy(g_hbm, g_v)
    pltpu.sync_copy(y0_hbm.at[pl.ds(base, _RPW)], idx_v)
    pltpu.sync_copy(z_hbm.at[pl.ds(base, _RPW)], z_v)

    lanes = lax.iota(jnp.int32, _L)

    def group(gi, carry):
        rowv = gi * _L + lanes
        idxv = idx_v[pl.ds(gi * _L, _L)]
        acc = jnp.zeros((_L,), jnp.float32)
        for j in range(_D):
            jv = jnp.full((_L,), j, jnp.int32)
            zc = plsc.load_gather(z_v, [rowv, jv])
            gc = plsc.load_gather(g_v, [idxv, jv])
            acc = acc + zc * gc
        out_v[pl.ds(gi * _L, _L)] = acc
        return carry

    lax.fori_loop(0, _G, group, 0)
    pltpu.sync_copy(out_v, out_hbm.at[pl.ds(base, _RPW)])


@functools.cache
def _sc_call():
    return functools.partial(
        pl.kernel,
        out_type=jax.ShapeDtypeStruct((_B,), jnp.float32),
        mesh=plsc.VectorSubcoreMesh(
            core_axis_name="c", subcore_axis_name="s",
            num_cores=_NC, num_subcores=_NS),
        compiler_params=pltpu.CompilerParams(needs_layout_passes=False),
        scratch_types=[
            pltpu.VMEM((_RPW, _D), jnp.float32),
            pltpu.VMEM((4, _D), jnp.float32),
            pltpu.VMEM((_RPW,), jnp.int32),
            pltpu.VMEM((_RPW,), jnp.float32),
        ],
    )(_sc_body)


def kernel(z, y, gamma):
    y0 = y[:, 0].astype(jnp.int32)
    return _sc_call()(z, y0, gamma)


# speedup vs baseline: 2.4679x; 2.4679x over previous
import functools

import jax
import jax.numpy as jnp
from jax import lax
from jax.experimental import pallas as pl
from jax.experimental.pallas import tpu as pltpu
from jax.experimental.pallas import tpu_sc as plsc

_B = 16384
_D = 128
_NC, _NS, _L = 2, 16, 16
_NW = _NC * _NS
_RPW = _B // _NW
_G = _RPW // _L


def _sc_body(z_hbm, y0_hbm, g_hbm, out_hbm, z_v, g_v, idx_v, out_v):
    wid = lax.axis_index("s") * _NC + lax.axis_index("c")
    base = wid * _RPW
    pltpu.sync_copy(g_hbm, g_v)
    pltpu.sync_copy(y0_hbm.at[pl.ds(base, _RPW)], idx_v)
    pltpu.sync_copy(z_hbm.at[pl.ds(base, _RPW)], z_v)

    lanes = lax.iota(jnp.int32, _L)

    def group(gi):
        idxvec = idx_v[pl.ds(gi * _L, _L)]
        sums = jnp.zeros((_L,), jnp.float32)
        for r in range(_L):
            b = gi * _L + r
            idx = idxvec[r]
            p = [z_v[b, pl.ds(16 * j, 16)] * g_v[idx, pl.ds(16 * j, 16)]
                 for j in range(_D // 16)]
            s = (((p[0] + p[1]) + (p[2] + p[3]))
                 + ((p[4] + p[5]) + (p[6] + p[7])))
            sums = jnp.where(lanes == r, jnp.sum(s), sums)
        out_v[pl.ds(gi * _L, _L)] = sums

    plsc.parallel_loop(0, _G, 1)(group)
    pltpu.sync_copy(out_v, out_hbm.at[pl.ds(base, _RPW)])


@functools.cache
def _sc_call():
    return functools.partial(
        pl.kernel,
        out_type=jax.ShapeDtypeStruct((_B,), jnp.float32),
        mesh=plsc.VectorSubcoreMesh(
            core_axis_name="c", subcore_axis_name="s",
            num_cores=_NC, num_subcores=_NS),
        compiler_params=pltpu.CompilerParams(needs_layout_passes=False),
        scratch_types=[
            pltpu.VMEM((_RPW, _D), jnp.float32),
            pltpu.VMEM((4, _D), jnp.float32),
            pltpu.VMEM((_RPW,), jnp.int32),
            pltpu.VMEM((_RPW,), jnp.float32),
        ],
    )(_sc_body)


def kernel(z, y, gamma):
    y0 = y[:, 0].astype(jnp.int32)
    return _sc_call()(z, y0, gamma)


# speedup vs baseline: 2.8709x; 1.1633x over previous
import functools

import jax
import jax.numpy as jnp
from jax import lax
from jax.experimental import pallas as pl
from jax.experimental.pallas import tpu as pltpu
from jax.experimental.pallas import tpu_sc as plsc

_B = 16384
_D = 128
_NC, _NS, _L = 2, 16, 16
_NW = _NC * _NS
_RPW = _B // _NW
_G = _RPW // _L
_DP = _D + 3


def _sc_body(z_hbm, y0_hbm, g_hbm, out_hbm, z_v, g_v, idx_v, out_v):
    wid = lax.axis_index("s") * _NC + lax.axis_index("c")
    base = wid * _RPW
    pltpu.sync_copy(g_hbm, g_v)
    pltpu.sync_copy(y0_hbm.at[pl.ds(base, _RPW)], idx_v)
    pltpu.sync_copy(z_hbm.at[pl.ds(base, _RPW)], z_v)

    lanes = lax.iota(jnp.int32, _L)

    def group(gi):
        rowv = gi * _L + lanes
        idxvec = idx_v[pl.ds(gi * _L, _L)]
        jv = rowv & (_D - 1)
        acc = [jnp.zeros((_L,), jnp.float32) for _ in range(4)]
        for j in range(_D):
            zc = plsc.load_gather(z_v, [rowv, jv])
            gc = plsc.load_gather(g_v, [idxvec, jv])
            acc[j % 4] = acc[j % 4] + zc * gc
            jv = (jv + 1) & (_D - 1)
        out_v[pl.ds(gi * _L, _L)] = (acc[0] + acc[1]) + (acc[2] + acc[3])

    plsc.parallel_loop(0, _G, 1)(group)
    pltpu.sync_copy(out_v, out_hbm.at[pl.ds(base, _RPW)])


@functools.cache
def _sc_call():
    return functools.partial(
        pl.kernel,
        out_type=jax.ShapeDtypeStruct((_B,), jnp.float32),
        mesh=plsc.VectorSubcoreMesh(
            core_axis_name="c", subcore_axis_name="s",
            num_cores=_NC, num_subcores=_NS),
        compiler_params=pltpu.CompilerParams(needs_layout_passes=False),
        scratch_types=[
            pltpu.VMEM((_RPW, _D), jnp.float32),
            pltpu.VMEM((4, _D), jnp.float32),
            pltpu.VMEM((_RPW,), jnp.int32),
            pltpu.VMEM((_RPW,), jnp.float32),
        ],
    )(_sc_body)


def kernel(z, y, gamma):
    y0 = y[:, 0].astype(jnp.int32)
    return _sc_call()(z, y0, gamma)
